# baseline (speedup 1.0000x reference)
import functools

import jax
import jax.numpy as jnp
from jax import lax
from jax.experimental import pallas as pl
from jax.experimental.pallas import tpu as pltpu
from jax.experimental.pallas import tpu_sc as plsc

VOCAB = 1000000
HIDDEN = 64
OUT = 64
BATCH = 16384

NUM_CORES = 2
NUM_SUBCORES = 16
NUM_WORKERS = NUM_CORES * NUM_SUBCORES

TOTAL_IDX = 2 * BATCH
IDX_PER_WORKER = TOTAL_IDX // NUM_WORKERS
LINES = VOCAB // 2


@functools.cache
def _sc_gather_lines():
    mesh = plsc.VectorSubcoreMesh(core_axis_name="c", subcore_axis_name="s")

    @functools.partial(
        pl.kernel,
        mesh=mesh,
        out_type=jax.ShapeDtypeStruct((TOTAL_IDX, 2 * HIDDEN), jnp.float32),
        scratch_types=[
            pltpu.VMEM((IDX_PER_WORKER // 2,), jnp.int32),
            pltpu.VMEM((IDX_PER_WORKER // 2, 2 * HIDDEN), jnp.float32),
            pltpu.SemaphoreType.DMA,
        ],
    )
    def gather(lines_hbm, lidx_hbm, out_hbm, lidx_v, rows_v, sem):
        wid = lax.axis_index("s") * NUM_CORES + lax.axis_index("c")
        half = IDX_PER_WORKER // 2
        for r in range(2):
            base = wid * IDX_PER_WORKER + r * half
            pltpu.sync_copy(lidx_hbm.at[pl.ds(base, half)], lidx_v)
            pltpu.async_copy(lines_hbm.at[lidx_v], rows_v, sem).wait()
            pltpu.sync_copy(rows_v, out_hbm.at[pl.ds(base, half)])

    return gather


def _tc_body(g1_ref, g2_ref, p1_ref, p2_ref, w_ref, b_ref, o_ref):
    g1, g2 = g1_ref[...], g2_ref[...]
    r1 = jnp.where(p1_ref[...] == 1, g1[:, HIDDEN:], g1[:, :HIDDEN])
    r2 = jnp.where(p2_ref[...] == 1, g2[:, HIDDEN:], g2[:, :HIDDEN])
    o_ref[...] = (
        jnp.dot(r1 - r2, w_ref[...], preferred_element_type=jnp.float32)
        + b_ref[...]
    )


@functools.cache
def _tc_linear():
    grid = 16
    blk = BATCH // grid
    return pl.pallas_call(
        _tc_body,
        grid=(grid,),
        in_specs=[
            pl.BlockSpec((blk, 2 * HIDDEN), lambda i: (i, 0)),
            pl.BlockSpec((blk, 2 * HIDDEN), lambda i: (i + grid, 0)),
            pl.BlockSpec((blk, 1), lambda i: (i, 0)),
            pl.BlockSpec((blk, 1), lambda i: (i + grid, 0)),
            pl.BlockSpec((HIDDEN, OUT), lambda i: (0, 0)),
            pl.BlockSpec((1, OUT), lambda i: (0, 0)),
        ],
        out_specs=pl.BlockSpec((blk, OUT), lambda i: (i, 0)),
        out_shape=jax.ShapeDtypeStruct((BATCH, OUT), jnp.float32),
    )


def kernel(sentences_seq, sentence_lengths, entity1_index, entity2_index,
           position_to_entity1_batch, position_to_entity2_batch,
           emb_table, W, b):
    idx = jnp.concatenate(
        [entity1_index.reshape(-1), entity2_index.reshape(-1)]
    ).astype(jnp.int32)
    lines = emb_table.reshape(LINES, 2 * HIDDEN)
    gathered = _sc_gather_lines()(lines, idx // 2)
    parity = (idx % 2).reshape(TOTAL_IDX, 1)
    return _tc_linear()(
        gathered, gathered, parity, parity, W, b.reshape(1, OUT)
    )


# speedup vs baseline: 2.4547x; 2.4547x over previous
import functools

import jax
import jax.numpy as jnp
from jax import lax
from jax.experimental import pallas as pl
from jax.experimental.pallas import tpu as pltpu
from jax.experimental.pallas import tpu_sc as plsc

VOCAB = 1000000
HIDDEN = 64
OUT = 64
BATCH = 16384

NUM_CORES = 2
NUM_SUBCORES = 16
NUM_WORKERS = NUM_CORES * NUM_SUBCORES

TOTAL_IDX = 2 * BATCH
N_TCOLS = (VOCAB + 127) // 128
COLS_PER_W = (N_TCOLS + NUM_WORKERS - 1) // NUM_WORKERS
L = 16

CAP = 1792
NGROUPS = 16
GCAP = 192
NBUF = 4
RING = 8
SENTINEL = 0x7FFFFFFF


@functools.cache
def _sc_stream_gather():
    mesh = plsc.VectorSubcoreMesh(core_axis_name="c", subcore_axis_name="s")

    @functools.partial(
        pl.kernel,
        mesh=mesh,
        compiler_params=pltpu.CompilerParams(needs_layout_passes=False),
        out_type=jax.ShapeDtypeStruct((TOTAL_IDX * HIDDEN,), jnp.float32),
        scratch_types=[
            pltpu.VMEM((TOTAL_IDX,), jnp.int32),
            pltpu.VMEM((NBUF, HIDDEN, 128), jnp.float32),
            pltpu.VMEM((CAP,), jnp.int32),
            pltpu.VMEM((NGROUPS * GCAP,), jnp.int32),
            pltpu.VMEM((RING * HIDDEN,), jnp.float32),
            pltpu.SemaphoreType.DMA,
            pltpu.SemaphoreType.DMA,
        ],
    )
    def gather(tableT_hbm, idx_hbm, out_hbm, idx_v, colbuf, lv, gl, rows_v,
               sem_col, sem_out):
        wid = lax.axis_index("s") * NUM_CORES + lax.axis_index("c")
        c_lo = wid * COLS_PER_W
        n_cols = jnp.minimum(N_TCOLS - c_lo, COLS_PER_W)
        lanes = lax.iota(jnp.int32, L)

        pltpu.sync_copy(idx_hbm, idx_v)

        def scan_vec(i, off):
            va = idx_v[pl.ds(i * L, L)]
            tc = lax.shift_right_logical(va, 7)
            c_rel = tc - c_lo
            m = (c_rel >= 0) & (c_rel < n_cols)
            dest = i * L + lanes
            pack = (
                lax.shift_left(c_rel, 22)
                | lax.shift_left(dest, 7)
                | (va & 127)
            )
            pref = lax.cumsum(m.astype(jnp.int32))
            plsc.store_scatter(lv, [off + pref - 1], pack, mask=m)
            return off + jnp.max(pref)

        n_match = lax.fori_loop(0, TOTAL_IDX // L, scan_vec, 0)

        def fill(i, x):
            gl[pl.ds(i * L, L)] = jnp.full((L,), SENTINEL, jnp.int32)
            return x

        lax.fori_loop(0, NGROUPS * GCAP // L, fill, 0)

        def split_vec(i, offs):
            pack = lv[pl.ds(i * L, L)]
            valid = (i * L + lanes) < n_match
            grp = lax.shift_right_logical(pack, 26)
            new_offs = []
            for g in range(NGROUPS):
                mg = valid & (grp == g)
                pref = lax.cumsum(mg.astype(jnp.int32))
                plsc.store_scatter(
                    gl, [g * GCAP + offs[g] + pref - 1], pack, mask=mg)
                new_offs.append(offs[g] + jnp.max(pref))
            return tuple(new_offs)

        lax.fori_loop(0, (CAP + L - 1) // L, split_vec,
                      (jnp.int32(0),) * NGROUPS)

        def fetch(c_rel):
            pltpu.make_async_copy(
                tableT_hbm.at[:, pl.ds((c_lo + c_rel) * 128, 128)],
                colbuf.at[lax.rem(c_rel, NBUF)], sem_col,
            ).start()

        for p in range(NBUF - 1):
            fetch(jnp.int32(p))

        def col_step(c_rel, n_out):
            buf = lax.rem(c_rel, NBUF)
            pltpu.make_async_copy(
                tableT_hbm.at[:, pl.ds(0, 128)], colbuf.at[buf], sem_col
            ).wait()

            @pl.when(c_rel + NBUF - 1 < n_cols)
            def _():
                fetch(c_rel + NBUF - 1)

            g = lax.shift_right_logical(c_rel, 4)

            def scan_bucket(i, n_out):
                pk16 = gl[pl.ds(g * GCAP + i * L, L)]
                m = lax.shift_right_logical(pk16, 22) == c_rel

                def extract_one(state):
                    m, n_out = state
                    lane0 = jnp.max(plsc.all_reduce_ffs(m))

                    @pl.when(n_out >= RING)
                    def _():
                        pltpu.make_async_copy(
                            rows_v.at[pl.ds(0, HIDDEN)],
                            out_hbm.at[pl.ds(0, HIDDEN)], sem_out,
                        ).wait()

                    pk = jnp.max(jnp.where(lanes == lane0, pk16, 0))
                    r = pk & 127
                    dest = lax.shift_right_logical(pk, 7) & 32767
                    slot = lax.rem(n_out, RING)
                    for k in range(HIDDEN // L):
                        seg = plsc.load_gather(
                            colbuf,
                            [jnp.full((L,), buf, jnp.int32),
                             lanes + k * L,
                             jnp.full((L,), r, jnp.int32)])
                        rows_v[pl.ds(slot * HIDDEN + k * L, L)] = seg
                    pltpu.make_async_copy(
                        rows_v.at[pl.ds(slot * HIDDEN, HIDDEN)],
                        out_hbm.at[pl.ds(dest * HIDDEN, HIDDEN)],
                        sem_out,
                    ).start()
                    return m & (lanes != lane0), n_out + 1

                def any_left(state):
                    m, _ = state
                    return jnp.max(plsc.all_reduce_population_count(m)) > 0

                m, n_out = lax.while_loop(any_left, extract_one, (m, n_out))
                return n_out

            return lax.fori_loop(0, GCAP // L, scan_bucket, n_out)

        n_out = lax.fori_loop(0, n_cols, col_step, 0)

        def drain(i, x):
            pltpu.make_async_copy(
                rows_v.at[pl.ds(0, HIDDEN)],
                out_hbm.at[pl.ds(0, HIDDEN)], sem_out,
            ).wait()
            return x

        lax.fori_loop(0, jnp.minimum(n_out, RING), drain, 0)

    return gather


def _tc_body(r1_ref, r2_ref, w_ref, b_ref, o_ref):
    rel = r1_ref[...] - r2_ref[...]
    o_ref[...] = (
        jnp.dot(rel, w_ref[...], preferred_element_type=jnp.float32)
        + b_ref[...]
    )


@functools.cache
def _tc_linear():
    grid = 16
    blk = BATCH // grid
    return pl.pallas_call(
        _tc_body,
        grid=(grid,),
        in_specs=[
            pl.BlockSpec((blk, HIDDEN), lambda i: (i, 0)),
            pl.BlockSpec((blk, HIDDEN), lambda i: (i + grid, 0)),
            pl.BlockSpec((HIDDEN, OUT), lambda i: (0, 0)),
            pl.BlockSpec((1, OUT), lambda i: (0, 0)),
        ],
        out_specs=pl.BlockSpec((blk, OUT), lambda i: (i, 0)),
        out_shape=jax.ShapeDtypeStruct((BATCH, OUT), jnp.float32),
    )


def kernel(sentences_seq, sentence_lengths, entity1_index, entity2_index,
           position_to_entity1_batch, position_to_entity2_batch,
           emb_table, W, b):
    idx = jnp.concatenate(
        [entity1_index.reshape(-1), entity2_index.reshape(-1)]
    ).astype(jnp.int32)
    flat = _sc_stream_gather()(emb_table.T, idx)
    rows = flat.reshape(TOTAL_IDX, HIDDEN)
    return _tc_linear()(rows, rows, W, b.reshape(1, OUT))


# speedup vs baseline: 3.0486x; 1.2419x over previous
import functools

import jax
import jax.numpy as jnp
from jax import lax
from jax.experimental import pallas as pl
from jax.experimental.pallas import tpu as pltpu
from jax.experimental.pallas import tpu_sc as plsc

VOCAB = 1000000
HIDDEN = 64
OUT = 64
BATCH = 16384

NUM_CORES = 2
NUM_SUBCORES = 16
NUM_WORKERS = NUM_CORES * NUM_SUBCORES

TOTAL_IDX = 2 * BATCH
N_TCOLS = (VOCAB + 127) // 128
COLS_PER_W = (N_TCOLS + NUM_WORKERS - 1) // NUM_WORKERS
L = 16

CAP = 1792
NGROUPS = 16
GCAP = 192
NBUF = 4
RING = 8
SENTINEL = 0x7FFFFFFF


@functools.cache
def _sc_stream_gather():
    mesh = plsc.VectorSubcoreMesh(core_axis_name="c", subcore_axis_name="s")

    @functools.partial(
        pl.kernel,
        mesh=mesh,
        compiler_params=pltpu.CompilerParams(needs_layout_passes=False),
        out_type=jax.ShapeDtypeStruct((TOTAL_IDX * HIDDEN,), jnp.float32),
        scratch_types=[
            pltpu.VMEM((TOTAL_IDX,), jnp.int32),
            pltpu.VMEM((NBUF, HIDDEN, 128), jnp.float32),
            pltpu.VMEM((CAP,), jnp.int32),
            pltpu.VMEM((NGROUPS * GCAP,), jnp.int32),
            pltpu.VMEM((RING * HIDDEN,), jnp.float32),
            pltpu.SemaphoreType.DMA,
            pltpu.SemaphoreType.DMA,
        ],
    )
    def gather(tableT_hbm, idx_hbm, out_hbm, idx_v, colbuf, lv, gl, rows_v,
               sem_col, sem_out):
        wid = lax.axis_index("s") * NUM_CORES + lax.axis_index("c")
        c_lo = wid * COLS_PER_W
        n_cols = jnp.minimum(N_TCOLS - c_lo, COLS_PER_W)
        lanes = lax.iota(jnp.int32, L)

        pltpu.sync_copy(idx_hbm, idx_v)

        def scan_vec(i, off):
            va = idx_v[pl.ds(i * L, L)]
            tc = lax.shift_right_logical(va, 7)
            c_rel = tc - c_lo
            m = (c_rel >= 0) & (c_rel < n_cols)
            dest = i * L + lanes
            pack = (
                lax.shift_left(c_rel, 22)
                | lax.shift_left(dest, 7)
                | (va & 127)
            )
            pref = lax.cumsum(m.astype(jnp.int32))
            plsc.store_scatter(lv, [off + pref - 1], pack, mask=m)
            return off + jnp.max(pref)

        n_match = lax.fori_loop(0, TOTAL_IDX // L, scan_vec, 0)

        def fill(i, x):
            gl[pl.ds(i * L, L)] = jnp.full((L,), SENTINEL, jnp.int32)
            return x

        lax.fori_loop(0, NGROUPS * GCAP // L, fill, 0)

        def split_vec(i, offs):
            pack = lv[pl.ds(i * L, L)]
            valid = (i * L + lanes) < n_match
            grp = lax.shift_right_logical(pack, 26)
            new_offs = []
            for g in range(NGROUPS):
                mg = valid & (grp == g)
                pref = lax.cumsum(mg.astype(jnp.int32))
                plsc.store_scatter(
                    gl, [g * GCAP + offs[g] + pref - 1], pack, mask=mg)
                new_offs.append(offs[g] + jnp.max(pref))
            return tuple(new_offs)

        goffs = lax.fori_loop(
            0, lax.shift_right_logical(n_match + L - 1, 4), split_vec,
            (jnp.int32(0),) * NGROUPS)

        def fetch(c_rel):
            pltpu.make_async_copy(
                tableT_hbm.at[:, pl.ds((c_lo + c_rel) * 128, 128)],
                colbuf.at[lax.rem(c_rel, NBUF)], sem_col,
            ).start()

        for p in range(NBUF - 1):
            fetch(jnp.int32(p))

        def col_step(c_rel, n_out):
            buf = lax.rem(c_rel, NBUF)
            pltpu.make_async_copy(
                tableT_hbm.at[:, pl.ds(0, 128)], colbuf.at[buf], sem_col
            ).wait()

            @pl.when(c_rel + NBUF - 1 < n_cols)
            def _():
                fetch(c_rel + NBUF - 1)

            g = lax.shift_right_logical(c_rel, 4)
            cnt_g = goffs[NGROUPS - 1]
            for gg in range(NGROUPS - 1):
                cnt_g = jnp.where(g == gg, goffs[gg], cnt_g)

            def scan_bucket(i, n_out):
                pk16 = gl[pl.ds(g * GCAP + i * L, L)]
                m = lax.shift_right_logical(pk16, 22) == c_rel
                nm = jnp.max(plsc.all_reduce_population_count(m))

                def extract_one(j, state):
                    m, n_out = state
                    lane0 = jnp.max(plsc.all_reduce_ffs(m))

                    @pl.when(n_out >= RING)
                    def _():
                        pltpu.make_async_copy(
                            rows_v.at[pl.ds(0, HIDDEN)],
                            out_hbm.at[pl.ds(0, HIDDEN)], sem_out,
                        ).wait()

                    pk = jnp.max(jnp.where(lanes == lane0, pk16, 0))
                    r = pk & 127
                    dest = lax.shift_right_logical(pk, 7) & 32767
                    slot = lax.rem(n_out, RING)
                    for k in range(HIDDEN // L):
                        seg = plsc.load_gather(
                            colbuf,
                            [jnp.full((L,), buf, jnp.int32),
                             lanes + k * L,
                             jnp.full((L,), r, jnp.int32)])
                        rows_v[pl.ds(slot * HIDDEN + k * L, L)] = seg
                    pltpu.make_async_copy(
                        rows_v.at[pl.ds(slot * HIDDEN, HIDDEN)],
                        out_hbm.at[pl.ds(dest * HIDDEN, HIDDEN)],
                        sem_out,
                    ).start()
                    return m & (lanes != lane0), n_out + 1

                m, n_out = lax.fori_loop(0, nm, extract_one, (m, n_out))
                return n_out

            return lax.fori_loop(
                0, lax.shift_right_logical(cnt_g + L - 1, 4),
                scan_bucket, n_out)

        n_out = lax.fori_loop(0, n_cols, col_step, 0)

        def drain(i, x):
            pltpu.make_async_copy(
                rows_v.at[pl.ds(0, HIDDEN)],
                out_hbm.at[pl.ds(0, HIDDEN)], sem_out,
            ).wait()
            return x

        lax.fori_loop(0, jnp.minimum(n_out, RING), drain, 0)

    return gather


def _tc_body(r1_ref, r2_ref, w_ref, b_ref, o_ref):
    rel = r1_ref[...] - r2_ref[...]
    o_ref[...] = (
        jnp.dot(rel, w_ref[...], preferred_element_type=jnp.float32)
        + b_ref[...]
    )


@functools.cache
def _tc_linear():
    grid = 16
    blk = BATCH // grid
    return pl.pallas_call(
        _tc_body,
        grid=(grid,),
        in_specs=[
            pl.BlockSpec((blk, HIDDEN), lambda i: (i, 0)),
            pl.BlockSpec((blk, HIDDEN), lambda i: (i + grid, 0)),
            pl.BlockSpec((HIDDEN, OUT), lambda i: (0, 0)),
            pl.BlockSpec((1, OUT), lambda i: (0, 0)),
        ],
        out_specs=pl.BlockSpec((blk, OUT), lambda i: (i, 0)),
        out_shape=jax.ShapeDtypeStruct((BATCH, OUT), jnp.float32),
    )


def kernel(sentences_seq, sentence_lengths, entity1_index, entity2_index,
           position_to_entity1_batch, position_to_entity2_batch,
           emb_table, W, b):
    idx = jnp.concatenate(
        [entity1_index.reshape(-1), entity2_index.reshape(-1)]
    ).astype(jnp.int32)
    flat = _sc_stream_gather()(emb_table.T, idx)
    rows = flat.reshape(TOTAL_IDX, HIDDEN)
    return _tc_linear()(rows, rows, W, b.reshape(1, OUT))


# speedup vs baseline: 3.2402x; 1.0628x over previous
import functools

import jax
import jax.numpy as jnp
from jax import lax
from jax.experimental import pallas as pl
from jax.experimental.pallas import tpu as pltpu
from jax.experimental.pallas import tpu_sc as plsc

VOCAB = 1000000
HIDDEN = 64
OUT = 64
BATCH = 16384

NUM_CORES = 2
NUM_SUBCORES = 16
NUM_WORKERS = NUM_CORES * NUM_SUBCORES

TOTAL_IDX = 2 * BATCH
N_TCOLS = (VOCAB + 127) // 128
COLS_PER_W = (N_TCOLS + NUM_WORKERS - 1) // NUM_WORKERS
L = 16

CAP = 1792
NGROUPS = 16
GCAP = 192
NBUF = 4
RING = 8
SENTINEL = 0x7FFFFFFF


@functools.cache
def _sc_stream_gather():
    mesh = plsc.VectorSubcoreMesh(core_axis_name="c", subcore_axis_name="s")

    @functools.partial(
        pl.kernel,
        mesh=mesh,
        compiler_params=pltpu.CompilerParams(needs_layout_passes=False),
        out_type=jax.ShapeDtypeStruct((TOTAL_IDX * HIDDEN,), jnp.float32),
        scratch_types=[
            pltpu.VMEM((TOTAL_IDX,), jnp.int32),
            pltpu.VMEM((NBUF, HIDDEN, 128), jnp.float32),
            pltpu.VMEM((CAP,), jnp.int32),
            pltpu.VMEM((NGROUPS * GCAP,), jnp.int32),
            pltpu.VMEM((RING * HIDDEN,), jnp.float32),
            pltpu.VMEM((2 * L,), jnp.int32),
            pltpu.SemaphoreType.DMA,
            pltpu.SemaphoreType.DMA,
        ],
    )
    def gather(tableT_hbm, idx_hbm, out_hbm, idx_v, colbuf, lv, gl, rows_v,
               stg, sem_col, sem_out):
        wid = lax.axis_index("s") * NUM_CORES + lax.axis_index("c")
        c_lo = wid * COLS_PER_W
        n_cols = jnp.minimum(N_TCOLS - c_lo, COLS_PER_W)
        lanes = lax.iota(jnp.int32, L)

        pltpu.sync_copy(idx_hbm, idx_v)

        def scan_vec(i, off):
            va = idx_v[pl.ds(i * L, L)]
            tc = lax.shift_right_logical(va, 7)
            c_rel = tc - c_lo
            m = (c_rel >= 0) & (c_rel < n_cols)
            dest = i * L + lanes
            pack = (
                lax.shift_left(c_rel, 22)
                | lax.shift_left(dest, 7)
                | (va & 127)
            )
            pref = lax.cumsum(m.astype(jnp.int32))
            plsc.store_scatter(lv, [off + pref - 1], pack, mask=m)
            return off + pref[L - 1]

        n_match = lax.fori_loop(0, TOTAL_IDX // L, scan_vec, 0)

        def fill(i, x):
            gl[pl.ds(i * L, L)] = jnp.full((L,), SENTINEL, jnp.int32)
            return x

        lax.fori_loop(0, NGROUPS * GCAP // L, fill, 0)

        def split_vec(i, offs):
            pack = lv[pl.ds(i * L, L)]
            valid = (i * L + lanes) < n_match
            grp = lax.shift_right_logical(pack, 26)
            new_offs = []
            for g in range(NGROUPS):
                mg = valid & (grp == g)
                pref = lax.cumsum(mg.astype(jnp.int32))
                plsc.store_scatter(
                    gl, [g * GCAP + offs[g] + pref - 1], pack, mask=mg)
                new_offs.append(offs[g] + pref[L - 1])
            return tuple(new_offs)

        goffs = lax.fori_loop(
            0, lax.shift_right_logical(n_match + L - 1, 4), split_vec,
            (jnp.int32(0),) * NGROUPS)

        def fetch(c_rel):
            pltpu.make_async_copy(
                tableT_hbm.at[:, pl.ds((c_lo + c_rel) * 128, 128)],
                colbuf.at[lax.rem(c_rel, NBUF)], sem_col,
            ).start()

        for p in range(NBUF - 1):
            fetch(jnp.int32(p))

        def col_step(c_rel, n_out):
            buf = lax.rem(c_rel, NBUF)
            pltpu.make_async_copy(
                tableT_hbm.at[:, pl.ds(0, 128)], colbuf.at[buf], sem_col
            ).wait()

            @pl.when(c_rel + NBUF - 1 < n_cols)
            def _():
                fetch(c_rel + NBUF - 1)

            g = lax.shift_right_logical(c_rel, 4)
            cnt_g = goffs[NGROUPS - 1]
            for gg in range(NGROUPS - 1):
                cnt_g = jnp.where(g == gg, goffs[gg], cnt_g)

            def scan_bucket(i, n_out):
                pk16 = gl[pl.ds(g * GCAP + i * L, L)]
                m = lax.shift_right_logical(pk16, 22) == c_rel
                nm = plsc.all_reduce_population_count(m)[0]
                stg[pl.ds(0, L)] = pk16

                def extract_one(j, state):
                    m, n_out = state
                    lane0 = plsc.all_reduce_ffs(m)[0]

                    @pl.when(n_out >= RING)
                    def _():
                        pltpu.make_async_copy(
                            rows_v.at[pl.ds(0, HIDDEN)],
                            out_hbm.at[pl.ds(0, HIDDEN)], sem_out,
                        ).wait()

                    pk = stg[pl.ds(lane0, L)][0]
                    r = pk & 127
                    dest = lax.shift_right_logical(pk, 7) & 32767
                    slot = lax.rem(n_out, RING)
                    for k in range(HIDDEN // L):
                        seg = plsc.load_gather(
                            colbuf,
                            [jnp.full((L,), buf, jnp.int32),
                             lanes + k * L,
                             jnp.full((L,), r, jnp.int32)])
                        rows_v[pl.ds(slot * HIDDEN + k * L, L)] = seg
                    pltpu.make_async_copy(
                        rows_v.at[pl.ds(slot * HIDDEN, HIDDEN)],
                        out_hbm.at[pl.ds(dest * HIDDEN, HIDDEN)],
                        sem_out,
                    ).start()
                    return m & (lanes != lane0), n_out + 1

                m, n_out = lax.fori_loop(0, nm, extract_one, (m, n_out))
                return n_out

            return lax.fori_loop(
                0, lax.shift_right_logical(cnt_g + L - 1, 4),
                scan_bucket, n_out)

        n_out = lax.fori_loop(0, n_cols, col_step, 0)

        def drain(i, x):
            pltpu.make_async_copy(
                rows_v.at[pl.ds(0, HIDDEN)],
                out_hbm.at[pl.ds(0, HIDDEN)], sem_out,
            ).wait()
            return x

        lax.fori_loop(0, jnp.minimum(n_out, RING), drain, 0)

    return gather


def _tc_body(r1_ref, r2_ref, w_ref, b_ref, o_ref):
    rel = r1_ref[...] - r2_ref[...]
    o_ref[...] = (
        jnp.dot(rel, w_ref[...], preferred_element_type=jnp.float32)
        + b_ref[...]
    )


@functools.cache
def _tc_linear():
    grid = 16
    blk = BATCH // grid
    return pl.pallas_call(
        _tc_body,
        grid=(grid,),
        in_specs=[
            pl.BlockSpec((blk, HIDDEN), lambda i: (i, 0)),
            pl.BlockSpec((blk, HIDDEN), lambda i: (i + grid, 0)),
            pl.BlockSpec((HIDDEN, OUT), lambda i: (0, 0)),
            pl.BlockSpec((1, OUT), lambda i: (0, 0)),
        ],
        out_specs=pl.BlockSpec((blk, OUT), lambda i: (i, 0)),
        out_shape=jax.ShapeDtypeStruct((BATCH, OUT), jnp.float32),
    )


def kernel(sentences_seq, sentence_lengths, entity1_index, entity2_index,
           position_to_entity1_batch, position_to_entity2_batch,
           emb_table, W, b):
    idx = jnp.concatenate(
        [entity1_index.reshape(-1), entity2_index.reshape(-1)]
    ).astype(jnp.int32)
    flat = _sc_stream_gather()(emb_table.T, idx)
    rows = flat.reshape(TOTAL_IDX, HIDDEN)
    return _tc_linear()(rows, rows, W, b.reshape(1, OUT))
